# baseline (speedup 1.0000x reference)
import functools

import jax
import jax.numpy as jnp
from jax import lax
from jax.experimental import pallas as pl
from jax.experimental.pallas import tpu as pltpu
from jax.experimental.pallas import tpu_sc as plsc

NUM_CORES = 2
NUM_SUBCORES = 16
NUM_WORKERS = NUM_CORES * NUM_SUBCORES
LANES = 16
OUTP = 8


@functools.lru_cache(maxsize=None)
def _build_table_collapse(V, D, S):
    R = 8192
    grid = (V + R - 1) // R

    def body(table_ref, wt_ref, out_ref):
        out_ref[...] = jnp.dot(
            table_ref[...], wt_ref[...], preferred_element_type=jnp.float32
        ) * (1.0 / S)

    return pl.pallas_call(
        body,
        grid=(grid,),
        in_specs=[
            pl.BlockSpec((R, D), lambda i: (i, 0)),
            pl.BlockSpec((D, OUTP), lambda i: (0, 0)),
        ],
        out_specs=pl.BlockSpec((R, OUTP), lambda i: (i, 0)),
        out_shape=jax.ShapeDtypeStruct((V, OUTP), jnp.float32),
    )


@functools.lru_cache(maxsize=None)
def _build_gather_pool(S, B, V):
    assert B % NUM_WORKERS == 0
    bw = B // NUM_WORKERS

    mesh = plsc.VectorSubcoreMesh(
        core_axis_name="c", subcore_axis_name="s",
        num_cores=NUM_CORES, num_subcores=NUM_SUBCORES)

    @functools.partial(
        pl.kernel,
        out_type=jax.ShapeDtypeStruct((B, OUTP), jnp.float32),
        mesh=mesh,
        scratch_types=[
            pltpu.VMEM((S, bw), jnp.int32),
            pltpu.VMEM((bw, OUTP), jnp.float32),
            pltpu.SemaphoreType.DMA,
        ],
        compiler_params=pltpu.CompilerParams(use_tc_tiling_on_sc=False),
    )
    def gather_pool(small_hbm, text_hbm, binit_hbm, out_hbm,
                    idx_v, acc_v, sem):
        wid = lax.axis_index("s") * NUM_CORES + lax.axis_index("c")
        base = wid * bw

        pltpu.sync_copy(text_hbm.at[:, pl.ds(base, bw)], idx_v)
        pltpu.sync_copy(binit_hbm, acc_v)

        def fire_body(s, _):
            pltpu.async_copy(
                small_hbm.at[idx_v.at[s]], acc_v, sem, add=True)
            return 0
        lax.fori_loop(0, S, fire_body, 0)

        def drain_body(s, _):
            pltpu.make_async_copy(
                small_hbm.at[idx_v.at[0]], acc_v, sem).wait()
            return 0
        lax.fori_loop(0, S, drain_body, 0)

        pltpu.sync_copy(acc_v, out_hbm.at[pl.ds(base, bw)])

    return gather_pool


def kernel(text, table, W, b):
    S, B = text.shape
    V, D = table.shape
    OUT = W.shape[0]
    text = text.astype(jnp.int32)
    bw = B // NUM_WORKERS

    wt8 = jnp.concatenate(
        [W.T.astype(jnp.float32),
         jnp.zeros((D, OUTP - OUT), jnp.float32)], axis=1)
    small8 = _build_table_collapse(V, D, S)(table, wt8)

    binit = jnp.concatenate(
        [jnp.broadcast_to(b.astype(jnp.float32), (bw, OUT)),
         jnp.zeros((bw, OUTP - OUT), jnp.float32)], axis=1)

    out8 = _build_gather_pool(S, B, V)(small8, text, binit)
    return out8[:, :OUT]


# speedup vs baseline: 1.4048x; 1.4048x over previous
import functools

import jax
import jax.numpy as jnp
from jax import lax
from jax.experimental import pallas as pl
from jax.experimental.pallas import tpu as pltpu
from jax.experimental.pallas import tpu_sc as plsc

NUM_CORES = 2
NUM_SUBCORES = 16
NUM_WORKERS = NUM_CORES * NUM_SUBCORES
LANES = 16
OUTP = 8


@functools.lru_cache(maxsize=None)
def _build_table_collapse(V4, S):
    R = 10000
    assert V4 % R == 0
    grid = V4 // R

    def body(table_ref, wt_ref, out_ref):
        out_ref[...] = jnp.dot(
            table_ref[...], wt_ref[...], preferred_element_type=jnp.float32
        ) * (1.0 / S)

    return pl.pallas_call(
        body,
        grid=(grid,),
        in_specs=[
            pl.BlockSpec((R, 128), lambda i: (i, 0)),
            pl.BlockSpec((128, 4 * OUTP), lambda i: (0, 0)),
        ],
        out_specs=pl.BlockSpec((R, 4 * OUTP), lambda i: (i, 0)),
        out_shape=jax.ShapeDtypeStruct((V4, 4 * OUTP), jnp.float32),
    )


@functools.lru_cache(maxsize=None)
def _build_gather_pool(S, B, V):
    assert B % NUM_WORKERS == 0
    bw = B // NUM_WORKERS

    mesh = plsc.VectorSubcoreMesh(
        core_axis_name="c", subcore_axis_name="s",
        num_cores=NUM_CORES, num_subcores=NUM_SUBCORES)

    @functools.partial(
        pl.kernel,
        out_type=jax.ShapeDtypeStruct((B, OUTP), jnp.float32),
        mesh=mesh,
        scratch_types=[
            pltpu.VMEM((S, bw), jnp.int32),
            pltpu.VMEM((bw, OUTP), jnp.float32),
            pltpu.SemaphoreType.DMA,
        ],
        compiler_params=pltpu.CompilerParams(use_tc_tiling_on_sc=False),
    )
    def gather_pool(small_hbm, text_hbm, binit_hbm, out_hbm,
                    idx_v, acc_v, sem):
        wid = lax.axis_index("s") * NUM_CORES + lax.axis_index("c")
        base = wid * bw

        pltpu.sync_copy(text_hbm.at[:, pl.ds(base, bw)], idx_v)
        pltpu.sync_copy(binit_hbm, acc_v)

        def fire_body(s, _):
            pltpu.async_copy(
                small_hbm.at[idx_v.at[s]], acc_v, sem, add=True)
            return 0
        lax.fori_loop(0, S, fire_body, 0)

        def drain_body(s, _):
            pltpu.make_async_copy(
                small_hbm.at[idx_v.at[0]], acc_v, sem).wait()
            return 0
        lax.fori_loop(0, S, drain_body, 0)

        pltpu.sync_copy(acc_v, out_hbm.at[pl.ds(base, bw)])

    return gather_pool


def kernel(text, table, W, b):
    S, B = text.shape
    V, D = table.shape
    OUT = W.shape[0]
    text = text.astype(jnp.int32)
    bw = B // NUM_WORKERS

    wt8 = jnp.concatenate(
        [W.T.astype(jnp.float32),
         jnp.zeros((D, OUTP - OUT), jnp.float32)], axis=1)
    wt_big = jnp.kron(jnp.eye(4, dtype=jnp.float32), wt8)
    table4 = table.reshape(V // 4, 4 * D)
    small_big = _build_table_collapse(V // 4, S)(table4, wt_big)
    small8 = small_big.reshape(V, OUTP)

    binit = jnp.concatenate(
        [jnp.broadcast_to(b.astype(jnp.float32), (bw, OUT)),
         jnp.zeros((bw, OUTP - OUT), jnp.float32)], axis=1)

    out8 = _build_gather_pool(S, B, V)(small8, text, binit)
    return out8[:, :OUT]


# speedup vs baseline: 1.8099x; 1.2884x over previous
import functools

import jax
import jax.numpy as jnp
from jax import lax
from jax.experimental import pallas as pl
from jax.experimental.pallas import tpu as pltpu
from jax.experimental.pallas import tpu_sc as plsc

NUM_CORES = 2
NUM_SUBCORES = 16
NUM_WORKERS = NUM_CORES * NUM_SUBCORES
LANES = 16


@functools.lru_cache(maxsize=None)
def _build_gather_pool(S, B, V, D):
    assert B % NUM_WORKERS == 0
    bw = B // NUM_WORKERS
    assert D % LANES == 0
    dreg = D // LANES

    mesh = plsc.VectorSubcoreMesh(
        core_axis_name="c", subcore_axis_name="s",
        num_cores=NUM_CORES, num_subcores=NUM_SUBCORES)

    @functools.partial(
        pl.kernel,
        out_type=jax.ShapeDtypeStruct((B, D), jnp.float32),
        mesh=mesh,
        scratch_types=[
            pltpu.VMEM((S, bw), jnp.int32),
            pltpu.VMEM((bw, D), jnp.float32),
            pltpu.SemaphoreType.DMA,
        ],
        compiler_params=pltpu.CompilerParams(use_tc_tiling_on_sc=False),
    )
    def gather_pool(text_hbm, table_hbm, out_hbm, idx_v, acc_v, sem):
        wid = lax.axis_index("s") * NUM_CORES + lax.axis_index("c")
        base = wid * bw

        pltpu.sync_copy(text_hbm.at[:, pl.ds(base, bw)], idx_v)

        zero = jnp.zeros((LANES,), jnp.float32)
        def zero_body(i, _):
            for d in range(dreg):
                acc_v[i, pl.ds(d * LANES, LANES)] = zero
            return 0
        lax.fori_loop(0, bw, zero_body, 0)

        def fire_body(s, _):
            pltpu.async_copy(
                table_hbm.at[idx_v.at[s]], acc_v, sem, add=True)
            return 0
        lax.fori_loop(0, S, fire_body, 0)

        def drain_body(s, _):
            pltpu.make_async_copy(
                table_hbm.at[idx_v.at[0]], acc_v, sem).wait()
            return 0
        lax.fori_loop(0, S, drain_body, 0)

        pltpu.sync_copy(acc_v, out_hbm.at[pl.ds(base, bw)])

    return gather_pool


@functools.lru_cache(maxsize=None)
def _build_finish(B, D, OUT, S):
    def body(pooled_ref, wt_ref, b_ref, out_ref):
        pooled = pooled_ref[...] * (1.0 / S)
        out_ref[...] = (
            jnp.dot(pooled, wt_ref[...], preferred_element_type=jnp.float32)
            + b_ref[...]
        )

    return pl.pallas_call(
        body,
        out_shape=jax.ShapeDtypeStruct((B, OUT), jnp.float32),
    )


def kernel(text, table, W, b):
    S, B = text.shape
    V, D = table.shape
    OUT = W.shape[0]
    text = text.astype(jnp.int32)
    pooled = _build_gather_pool(S, B, V, D)(text, table)
    out = _build_finish(B, D, OUT, S)(pooled, W.T, b.reshape(1, OUT))
    return out


# speedup vs baseline: 1.8105x; 1.0003x over previous
import functools

import jax
import jax.numpy as jnp
from jax import lax
from jax.experimental import pallas as pl
from jax.experimental.pallas import tpu as pltpu
from jax.experimental.pallas import tpu_sc as plsc

NUM_CORES = 2
NUM_SUBCORES = 16
NUM_WORKERS = NUM_CORES * NUM_SUBCORES
LANES = 16


@functools.lru_cache(maxsize=None)
def _build_gather_pool(S, B, V, D):
    assert B % NUM_WORKERS == 0
    bw = B // NUM_WORKERS
    assert D % LANES == 0
    dreg = D // LANES

    mesh = plsc.VectorSubcoreMesh(
        core_axis_name="c", subcore_axis_name="s",
        num_cores=NUM_CORES, num_subcores=NUM_SUBCORES)

    @functools.partial(
        pl.kernel,
        out_type=jax.ShapeDtypeStruct((B, D), jnp.float32),
        mesh=mesh,
        scratch_types=[
            pltpu.VMEM((S, bw), jnp.int32),
            pltpu.VMEM((bw, D), jnp.float32),
            pltpu.SemaphoreType.DMA,
        ],
        compiler_params=pltpu.CompilerParams(use_tc_tiling_on_sc=False),
    )
    def gather_pool(text_hbm, table_hbm, out_hbm, idx_v, acc_v, sem):
        wid = lax.axis_index("s") * NUM_CORES + lax.axis_index("c")
        base = wid * bw

        pltpu.sync_copy(text_hbm.at[wid], idx_v)

        zero = jnp.zeros((LANES,), jnp.float32)
        def zero_body(i, _):
            for d in range(dreg):
                acc_v[i, pl.ds(d * LANES, LANES)] = zero
            return 0
        lax.fori_loop(0, bw, zero_body, 0)

        def fire_body(s, _):
            pltpu.async_copy(
                table_hbm.at[idx_v.at[s]], acc_v, sem, add=True)
            return 0
        lax.fori_loop(0, S, fire_body, 0)

        def drain_body(s, _):
            pltpu.make_async_copy(
                table_hbm.at[idx_v.at[0]], acc_v, sem).wait()
            return 0
        lax.fori_loop(0, S, drain_body, 0)

        pltpu.sync_copy(acc_v, out_hbm.at[pl.ds(base, bw)])

    return gather_pool


@functools.lru_cache(maxsize=None)
def _build_finish(B, D, OUT, S):
    def body(pooled_ref, wt_ref, b_ref, out_ref):
        pooled = pooled_ref[...] * (1.0 / S)
        out_ref[...] = (
            jnp.dot(pooled, wt_ref[...], preferred_element_type=jnp.float32)
            + b_ref[...]
        )

    return pl.pallas_call(
        body,
        out_shape=jax.ShapeDtypeStruct((B, OUT), jnp.float32),
    )


def kernel(text, table, W, b):
    S, B = text.shape
    V, D = table.shape
    OUT = W.shape[0]
    text = text.astype(jnp.int32)
    bw = B // NUM_WORKERS
    text_slabs = text.reshape(S, NUM_WORKERS, bw).transpose(1, 0, 2)
    pooled = _build_gather_pool(S, B, V, D)(text_slabs, table)
    out = _build_finish(B, D, OUT, S)(pooled, W.T, b.reshape(1, OUT))
    return out
